# baseline (speedup 1.0000x reference)
import jax
import jax.numpy as jnp
from jax import lax
from jax.experimental import pallas as pl
from jax.experimental.pallas import tpu as pltpu
from jax.experimental.pallas import tpu_sc as plsc

F = 26
V = 100000
D = 32
B = 16384

NC = 2
NS = 16
NW = NC * NS
NTASK = F * D
TPW = NTASK // NW
H0 = 50048
H1 = V - H0
BH = 8192
L = 16


def _embed_body(tab_hbm, cx_hbm, out_hbm, rowA, rowB, idx_v, outs,
                sem_ra, sem_rb, sem_out):
    wid = lax.axis_index("s") * NC + lax.axis_index("c")
    base = wid * TPW

    def pass0(bh):
        @plsc.parallel_loop(0, BH // L, unroll=8)
        def grp(j):
            vec = idx_v[pl.ds(bh * BH + j * L, L)]
            outs[pl.ds(j * L, L)] = plsc.load_gather(rowA, [vec], mask=vec < H0)

    def pass1(bh):
        @plsc.parallel_loop(0, BH // L, unroll=8)
        def grp(j):
            vec = idx_v[pl.ds(bh * BH + j * L, L)]
            m1 = vec >= H0
            g1 = plsc.load_gather(rowB, [vec - H0], mask=m1)
            prev = outs[pl.ds(j * L, L)]
            outs[pl.ds(j * L, L)] = jnp.where(m1, g1, prev)

    def srcA(f, d):
        return tab_hbm.at[f, d, pl.ds(0, H0)]

    def srcB(f, d):
        return tab_hbm.at[f, d, pl.ds(H0, H1)]

    f0 = base // D
    d0 = base % D
    pltpu.sync_copy(cx_hbm.at[f0], idx_v)
    pltpu.async_copy(srcA(f0, d0), rowA, sem_ra).wait()
    cpb = pltpu.async_copy(srcB(f0, d0), rowB, sem_rb)
    pass0(0)
    cpb.wait()
    pass1(0)
    pltpu.async_copy(outs, out_hbm.at[f0, d0, pl.ds(0, BH)], sem_out)
    pltpu.make_async_copy(out_hbm.at[f0, d0, pl.ds(BH, BH)], outs, sem_out).wait()
    pass0(1)
    f1 = (base + 1) // D
    d1 = (base + 1) % D
    pltpu.async_copy(srcA(f1, d1), rowA, sem_ra)
    pass1(1)
    pltpu.async_copy(srcB(f1, d1), rowB, sem_rb)
    pltpu.async_copy(outs, out_hbm.at[f0, d0, pl.ds(BH, BH)], sem_out)

    def task(t, carry):
        tid = base + t
        f = tid // D
        d = tid % D
        @pl.when(d == 0)
        def _():
            pltpu.sync_copy(cx_hbm.at[f], idx_v)

        pltpu.make_async_copy(srcA(f, d), rowA, sem_ra).wait()
        pltpu.make_async_copy(out_hbm.at[f, d, pl.ds(0, BH)], outs, sem_out).wait()
        pass0(0)
        pltpu.make_async_copy(srcB(f, d), rowB, sem_rb).wait()
        pass1(0)
        pltpu.async_copy(outs, out_hbm.at[f, d, pl.ds(0, BH)], sem_out)
        pltpu.make_async_copy(out_hbm.at[f, d, pl.ds(BH, BH)], outs, sem_out).wait()
        pass0(1)
        tid_n = tid + 1
        f_n = tid_n // D
        d_n = tid_n % D

        @pl.when(t < TPW - 1)
        def _pfa():
            pltpu.async_copy(srcA(f_n, d_n), rowA, sem_ra)

        pass1(1)

        @pl.when(t < TPW - 1)
        def _pfb():
            pltpu.async_copy(srcB(f_n, d_n), rowB, sem_rb)

        pltpu.async_copy(outs, out_hbm.at[f, d, pl.ds(BH, BH)], sem_out)
        return carry

    lax.fori_loop(1, TPW, task, 0)
    pltpu.make_async_copy(out_hbm.at[f0, d0, pl.ds(BH, BH)], outs, sem_out).wait()


@jax.jit
def kernel(cx, tables):
    cx_t = cx.T.astype(jnp.int32)
    tab_t = jnp.transpose(tables, (0, 2, 1))
    run = pl.kernel(
        _embed_body,
        out_type=jax.ShapeDtypeStruct((F, D, B), jnp.float32),
        mesh=plsc.VectorSubcoreMesh(core_axis_name="c", subcore_axis_name="s"),
        scratch_types=[
            pltpu.VMEM((H0,), jnp.float32),
            pltpu.VMEM((H1,), jnp.float32),
            pltpu.VMEM((B,), jnp.int32),
            pltpu.VMEM((BH,), jnp.float32),
            pltpu.SemaphoreType.DMA,
            pltpu.SemaphoreType.DMA,
            pltpu.SemaphoreType.DMA,
        ],
        compiler_params=pltpu.CompilerParams(use_tc_tiling_on_sc=True, needs_layout_passes=False),
    )
    out_t = run(tab_t, cx_t)
    return jnp.transpose(out_t, (2, 0, 1))
